# baseline (speedup 1.0000x reference)
import functools

import jax
import jax.numpy as jnp
from jax import lax
from jax.experimental import pallas as pl
from jax.experimental.pallas import tpu as pltpu

_C = 4
_RELA_TH = 0.5

_F32 = jnp.float32


def _relu(x):
    return jnp.maximum(x, 0.0)


def _dot(a, b):
    return jnp.dot(a, b, preferred_element_type=_F32)


def _dot_t(a, b):
    return lax.dot_general(a, b, (((0,), (0,)), ((), ())),
                           preferred_element_type=_F32)


def _lnk(x, g, b, eps=1e-5):
    m = jnp.mean(x, axis=-1, keepdims=True)
    v = jnp.mean((x - m) ** 2, axis=-1, keepdims=True)
    return (x - m) / jnp.sqrt(v + eps) * g + b


_PNAMES = (
    'preW', 'preb', 'preg', 'prebe',
    'subjW', 'subjb', 'objW', 'objb',
    'ps1W', 'ps1b', 'psg', 'psbe', 'ps2W', 'ps2b',
)
_NP = len(_PNAMES)

_WNAMES = (
    'pr1W', 'pr1b', 'prg', 'prbe', 'pr2W', 'pr2b',
    'rsfW', 'rsfb', 'rs2W', 'rs2b',
    'g1W', 'g1b', 'g2W', 'g2b', 'g3W', 'g3b',
)
_NW = len(_WNAMES)


def _pernode_body(bsz, n, d_m, img_ref, sbb_ref, *rest):
    w = {name: rest[i] for i, name in enumerate(_PNAMES)}
    xsop_ref, x_ref, mask_ref = rest[_NP:]
    img = img_ref[...]
    rowsum = jnp.sum(img, axis=1, keepdims=True)
    mask = (rowsum != 0.0).astype(_F32)
    x = _lnk(_relu(_dot(img, w['preW'][...]) + w['preb'][...]),
             w['preg'][...], w['prebe'][...])
    xs = _relu(_dot(x, w['subjW'][...]) + w['subjb'][...]) * mask
    xo = _relu(_dot(x, w['objW'][...]) + w['objb'][...]) * mask
    t1 = _lnk(_relu(_dot(sbb_ref[...], w['ps1W'][...]) + w['ps1b'][...]),
              w['psg'][...], w['psbe'][...])
    pself = _relu(_dot(t1, w['ps2W'][...]) + w['ps2b'][...]) * mask
    zblk = jnp.zeros((n, pself.shape[1]), _F32)
    for bi in range(bsz):
        r0 = bi * n
        xsop_ref[bi, :n, :d_m] = xo[r0:r0 + n]
        xsop_ref[bi, :n, d_m:] = zblk
        xsop_ref[bi, n:, :d_m] = xs[r0:r0 + n]
        xsop_ref[bi, n:, d_m:] = pself[r0:r0 + n]
        x_ref[bi] = x[r0:r0 + n]
        mask_ref[bi] = mask[r0:r0 + n]


def _body(n, s, d_rs, d_m, xsop_ref, x_ref, maskc_ref, bbox_ref, *rest):
    f = s * n
    w = {name: rest[i] for i, name in enumerate(_WNAMES)}
    out_ref, am_ref = rest[_NW], rest[_NW + 1]
    (s_maskr, s_relas, s_A, s_mt, s_T) = rest[_NW + 2:]
    c = pl.program_id(1)

    s0 = c * s
    ii = lax.broadcasted_iota(jnp.int32, (f, 1), 0)
    tt = ii % n
    ss = ii // n + s0
    coln = lax.broadcasted_iota(jnp.int32, (f, n), 1)

    @pl.when(c == 0)
    def _once():
        mask = maskc_ref[0]
        rr = lax.broadcasted_iota(jnp.int32, (n, n), 0)
        cc = lax.broadcasted_iota(jnp.int32, (n, n), 1)
        eye = (rr == cc).astype(_F32)
        s_maskr[...] = jnp.sum(eye * mask, axis=0, keepdims=True)
        am_ref[0, 0] = jnp.sum(eye * (1.0 - mask), axis=0, keepdims=True)
        T0 = (tt == coln).astype(_F32)
        s_T[...] = T0
        s_mt[...] = jnp.sum(T0 * s_maskr[...], axis=1, keepdims=True)
        s_relas[...] = jnp.zeros_like(s_relas)
        s_A[...] = jnp.zeros_like(s_A)

    T = s_T[...]
    Sx = (ss == coln).astype(_F32)
    dcol = (tt == ss).astype(_F32)
    mt = s_mt[...]
    ms = jnp.sum(Sx * s_maskr[...], axis=1, keepdims=True)

    bb = bbox_ref[0, 0]
    q = _lnk(_relu(_dot(bb, w['pr1W'][...]) + w['pr1b'][...]),
             w['prg'][...], w['prbe'][...])
    pp = _relu(_dot(q, w['pr2W'][...]) + w['pr2b'][...])
    sdidx = jnp.where(tt == ss, ss + n, tt)
    col2 = lax.broadcasted_iota(jnp.int32, (f, 2 * n), 1)
    SD = (sdidx == col2).astype(_F32)
    big = _dot(SD, xsop_ref[0])
    posr = big[:, d_m:] + (1.0 - dcol) * pp
    feat = jnp.concatenate([big[:, :d_m], posr], axis=1)

    hp = _relu(_dot(feat, w['rsfW'][...]) + w['rsfb'][...])
    h = hp[:, :d_rs]
    proj = hp[:, d_rs:]
    logits = _dot(h, w['rs2W'][...]) + w['rs2b'][...]
    pfl = jax.nn.sigmoid(logits) * ms * mt
    afl = (pfl > _RELA_TH).astype(_F32)
    s_relas[...] += _dot_t(Sx, pfl * proj)
    s_A[...] += _dot_t(Sx, afl * T)

    @pl.when(c == _C - 1)
    def _gcn_out():
        x = x_ref[0]
        relas = s_relas[...] * (1.0 / n)
        xx = jnp.concatenate([x, relas], axis=1)
        rr = lax.broadcasted_iota(jnp.int32, (n, n), 0)
        cc = lax.broadcasted_iota(jnp.int32, (n, n), 1)
        eye = (rr == cc).astype(_F32)
        A1 = jnp.maximum(s_A[...], eye)
        deg = jnp.sum(A1, axis=0, keepdims=True)
        dinv_r = jnp.where(deg > 0, 1.0 / jnp.sqrt(deg), 0.0)
        dinv_c = jnp.sum(eye * dinv_r, axis=1, keepdims=True)

        def gcn(y, W, b):
            yw = _dot(y, W)
            z = _dot_t(A1, dinv_c * yw)
            return dinv_c * z + b

        x1 = _relu(gcn(xx, w['g1W'][...], w['g1b'][...]))
        x2 = _relu(gcn(x1, w['g2W'][...], w['g2b'][...]))
        x3 = _relu(gcn(x2, w['g3W'][...], w['g3b'][...]))
        out_ref[0, 0] = x
        out_ref[0, 1] = x3


def kernel(images, selfbbox, bbox, params):
    b, n, d_in = images.shape
    s = n // _C
    f = s * n
    d_m = params['pre_W'].shape[1]
    d_ps = params['ps2_W'].shape[1]
    d_pair = params['rs1_W'].shape[0]
    d_rf = params['rf_W'].shape[1]
    p = params

    def row(v):
        return v.reshape(1, -1)

    pweights = [
        p['pre_W'], row(p['pre_b']), row(p['pre_g']), row(p['pre_be']),
        p['subj_W'], row(p['subj_b']), p['obj_W'], row(p['obj_b']),
        p['ps1_W'], row(p['ps1_b']), row(p['ps_g']), row(p['ps_be']),
        p['ps2_W'], row(p['ps2_b']),
    ]
    xsop, x_all, mask_all = pl.pallas_call(
        functools.partial(_pernode_body, b, n, d_m),
        out_shape=[
            jax.ShapeDtypeStruct((b, 2 * n, d_pair), _F32),
            jax.ShapeDtypeStruct((b, n, d_m), _F32),
            jax.ShapeDtypeStruct((b, n, 1), _F32),
        ],
    )(images.reshape(b * n, d_in), selfbbox.reshape(b * n, -1), *pweights)

    weights = [
        p['pr1_W'], row(p['pr1_b']), row(p['pr_g']), row(p['pr_be']),
        p['pr2_W'], row(p['pr2_b']),
        jnp.concatenate([p['rs1_W'], p['rf_W']], axis=1),
        jnp.concatenate([row(p['rs1_b']), row(p['rf_b'])], axis=1),
        p['rs2_W'], p['rs2_b'].reshape(1, 1),
        p['g1_W'], row(p['g1_b']), p['g2_W'], row(p['g2_b']),
        p['g3_W'], row(p['g3_b']),
    ]
    bbox_r = bbox.reshape(b, _C, f, bbox.shape[-1])

    def const_spec(shape):
        nd = len(shape)
        return pl.BlockSpec(shape, lambda bi, ci, _nd=nd: (0,) * _nd)

    in_specs = [
        pl.BlockSpec((1, 2 * n, d_pair), lambda bi, ci: (bi, 0, 0)),
        pl.BlockSpec((1, n, d_m), lambda bi, ci: (bi, 0, 0)),
        pl.BlockSpec((1, n, 1), lambda bi, ci: (bi, 0, 0)),
        pl.BlockSpec((1, 1, f, bbox.shape[-1]), lambda bi, ci: (bi, ci, 0, 0)),
    ] + [const_spec(wa.shape) for wa in weights]

    out_shape = [
        jax.ShapeDtypeStruct((b, 2, n, d_m), _F32),
        jax.ShapeDtypeStruct((b, 1, 1, n), _F32),
    ]
    out_specs = [
        pl.BlockSpec((1, 2, n, d_m), lambda bi, ci: (bi, 0, 0, 0)),
        pl.BlockSpec((1, 1, 1, n), lambda bi, ci: (bi, 0, 0, 0)),
    ]
    scratch_shapes = [
        pltpu.VMEM((1, n), _F32),
        pltpu.VMEM((n, d_rf), _F32),
        pltpu.VMEM((n, n), _F32),
        pltpu.VMEM((f, 1), _F32),
        pltpu.VMEM((f, n), _F32),
    ]

    body = functools.partial(_body, n, s, p['rs1_W'].shape[1], d_m)
    outs, am = pl.pallas_call(
        body,
        grid=(b, _C),
        in_specs=in_specs,
        out_specs=out_specs,
        out_shape=out_shape,
        scratch_shapes=scratch_shapes,
        compiler_params=pltpu.CompilerParams(
            dimension_semantics=("arbitrary", "arbitrary")),
    )(xsop, x_all, mask_all, bbox_r, *weights)
    return outs, am.astype(bool)


# speedup vs baseline: 1.0392x; 1.0392x over previous
import functools

import jax
import jax.numpy as jnp
from jax import lax
from jax.experimental import pallas as pl
from jax.experimental.pallas import tpu as pltpu

_C = 4
_RELA_TH = 0.5

_F32 = jnp.float32


def _relu(x):
    return jnp.maximum(x, 0.0)


def _dot(a, b):
    return jnp.dot(a, b, preferred_element_type=_F32)


def _dot_t(a, b):
    return lax.dot_general(a, b, (((0,), (0,)), ((), ())),
                           preferred_element_type=_F32)


def _lnk(x, g, b, eps=1e-5):
    m = jnp.mean(x, axis=-1, keepdims=True)
    v = jnp.mean((x - m) ** 2, axis=-1, keepdims=True)
    return (x - m) / jnp.sqrt(v + eps) * g + b


_WNAMES = (
    'preW', 'preb', 'preg', 'prebe',
    'subjW', 'subjb', 'objW', 'objb',
    'ps1W', 'ps1b', 'psg', 'psbe', 'ps2W', 'ps2b',
    'pr1W', 'pr1b', 'prg', 'prbe', 'pr2W', 'pr2b',
    'rsfW', 'rsfb', 'rs2W', 'rs2b',
    'g1W', 'g1b', 'g2W', 'g2b', 'g3W', 'g3b',
)
_NW = len(_WNAMES)


def _body(n, s, d_rs, d_m, img_ref, sbb_ref, bbox_ref, *rest):
    f = s * n
    w = {name: rest[i] for i, name in enumerate(_WNAMES)}
    out_ref, am_ref = rest[_NW], rest[_NW + 1]
    (s_x, s_maskc, s_maskr, s_xsop, s_relas, s_A, s_mt, s_T) = rest[_NW + 2:]
    c = pl.program_id(1)

    @pl.when(c == 0)
    def _pernode():
        img = img_ref[0]
        rowsum = jnp.sum(img, axis=1, keepdims=True)
        mask = (rowsum != 0.0).astype(_F32)
        s_maskc[...] = mask
        rr = lax.broadcasted_iota(jnp.int32, (n, n), 0)
        cc = lax.broadcasted_iota(jnp.int32, (n, n), 1)
        eye = (rr == cc).astype(_F32)
        s_maskr[...] = jnp.sum(eye * mask, axis=0, keepdims=True)
        am_ref[0, 0] = jnp.sum(eye * (1.0 - mask), axis=0, keepdims=True)

        x = _lnk(_relu(_dot(img, w['preW'][...]) + w['preb'][...]),
                 w['preg'][...], w['prebe'][...])
        s_x[...] = x
        xs = _relu(_dot(x, w['subjW'][...]) + w['subjb'][...]) * mask
        xo = _relu(_dot(x, w['objW'][...]) + w['objb'][...]) * mask
        sb = sbb_ref[0]
        t1 = _lnk(_relu(_dot(sb, w['ps1W'][...]) + w['ps1b'][...]),
                  w['psg'][...], w['psbe'][...])
        pself = _relu(_dot(t1, w['ps2W'][...]) + w['ps2b'][...]) * mask
        s_xsop[:n, :d_m] = xo
        s_xsop[:n, d_m:] = jnp.zeros_like(pself)
        s_xsop[n:, :d_m] = xs
        s_xsop[n:, d_m:] = pself
        s_relas[...] = jnp.zeros_like(s_relas)
        s_A[...] = jnp.zeros_like(s_A)

    s0 = c * s
    ii = lax.broadcasted_iota(jnp.int32, (f, 1), 0)
    tt = ii % n
    ss = ii // n + s0
    coln = lax.broadcasted_iota(jnp.int32, (f, n), 1)
    maskr = s_maskr[...]

    @pl.when(c == 0)
    def _t_once():
        T0 = (tt == coln).astype(_F32)
        s_T[...] = T0
        s_mt[...] = jnp.sum(T0 * maskr, axis=1, keepdims=True)

    T = s_T[...]
    Sx = (ss == coln).astype(_F32)
    dcol = (tt == ss).astype(_F32)
    mt = s_mt[...]
    ms = jnp.sum(Sx * maskr, axis=1, keepdims=True)

    bb = bbox_ref[0, 0]
    q = _lnk(_relu(_dot(bb, w['pr1W'][...]) + w['pr1b'][...]),
             w['prg'][...], w['prbe'][...])
    pp = _relu(_dot(q, w['pr2W'][...]) + w['pr2b'][...])
    sdidx = jnp.where(tt == ss, ss + n, tt)
    col2 = lax.broadcasted_iota(jnp.int32, (f, 2 * n), 1)
    SD = (sdidx == col2).astype(_F32)
    big = _dot(SD, s_xsop[...])
    posr = big[:, d_m:] + (1.0 - dcol) * pp
    feat = jnp.concatenate([big[:, :d_m], posr], axis=1)

    hp = _relu(_dot(feat, w['rsfW'][...]) + w['rsfb'][...])
    h = hp[:, :d_rs]
    proj = hp[:, d_rs:]
    logits = _dot(h, w['rs2W'][...]) + w['rs2b'][...]
    pfl = jax.nn.sigmoid(logits) * ms * mt
    afl = (pfl > _RELA_TH).astype(_F32)
    s_relas[...] += _dot_t(Sx, pfl * proj)
    s_A[...] += _dot_t(Sx, afl * T)

    @pl.when(c == _C - 1)
    def _gcn_out():
        x = s_x[...]
        relas = s_relas[...] * (1.0 / n)
        xx = jnp.concatenate([x, relas], axis=1)
        rr = lax.broadcasted_iota(jnp.int32, (n, n), 0)
        cc = lax.broadcasted_iota(jnp.int32, (n, n), 1)
        eye = (rr == cc).astype(_F32)
        A1 = jnp.maximum(s_A[...], eye)
        deg = jnp.sum(A1, axis=0, keepdims=True)
        dinv_r = jnp.where(deg > 0, 1.0 / jnp.sqrt(deg), 0.0)
        dinv_c = jnp.sum(eye * dinv_r, axis=1, keepdims=True)

        def gcn(y, W, b):
            yw = _dot(y, W)
            z = _dot_t(A1, dinv_c * yw)
            return dinv_c * z + b

        x1 = _relu(gcn(xx, w['g1W'][...], w['g1b'][...]))
        x2 = _relu(gcn(x1, w['g2W'][...], w['g2b'][...]))
        x3 = _relu(gcn(x2, w['g3W'][...], w['g3b'][...]))
        out_ref[0, 0] = x
        out_ref[0, 1] = x3


def kernel(images, selfbbox, bbox, params):
    b, n, d_in = images.shape
    s = n // _C
    f = s * n
    d_m = params['pre_W'].shape[1]
    p = params

    def row(v):
        return v.reshape(1, -1)

    weights = [
        p['pre_W'], row(p['pre_b']), row(p['pre_g']), row(p['pre_be']),
        p['subj_W'], row(p['subj_b']), p['obj_W'], row(p['obj_b']),
        p['ps1_W'], row(p['ps1_b']), row(p['ps_g']), row(p['ps_be']),
        p['ps2_W'], row(p['ps2_b']),
        p['pr1_W'], row(p['pr1_b']), row(p['pr_g']), row(p['pr_be']),
        p['pr2_W'], row(p['pr2_b']),
        jnp.concatenate([p['rs1_W'], p['rf_W']], axis=1),
        jnp.concatenate([row(p['rs1_b']), row(p['rf_b'])], axis=1),
        p['rs2_W'], p['rs2_b'].reshape(1, 1),
        p['g1_W'], row(p['g1_b']), p['g2_W'], row(p['g2_b']),
        p['g3_W'], row(p['g3_b']),
    ]
    bbox_r = bbox.reshape(b, _C, f, bbox.shape[-1])

    def const_spec(shape):
        nd = len(shape)
        return pl.BlockSpec(shape, lambda bi, ci, _nd=nd: (0,) * _nd)

    in_specs = [
        pl.BlockSpec((1, n, d_in), lambda bi, ci: (bi, 0, 0)),
        pl.BlockSpec((1, n, selfbbox.shape[-1]), lambda bi, ci: (bi, 0, 0)),
        pl.BlockSpec((1, 1, f, bbox.shape[-1]), lambda bi, ci: (bi, ci, 0, 0)),
    ] + [const_spec(wa.shape) for wa in weights]

    out_shape = [
        jax.ShapeDtypeStruct((b, 2, n, d_m), _F32),
        jax.ShapeDtypeStruct((b, 1, 1, n), _F32),
    ]
    out_specs = [
        pl.BlockSpec((1, 2, n, d_m), lambda bi, ci: (bi, 0, 0, 0)),
        pl.BlockSpec((1, 1, 1, n), lambda bi, ci: (bi, 0, 0, 0)),
    ]
    d_rf = p['rf_W'].shape[1]
    d_pair = p['rs1_W'].shape[0]
    scratch_shapes = [
        pltpu.VMEM((n, d_m), _F32),
        pltpu.VMEM((n, 1), _F32),
        pltpu.VMEM((1, n), _F32),
        pltpu.VMEM((2 * n, d_pair), _F32),
        pltpu.VMEM((n, d_rf), _F32),
        pltpu.VMEM((n, n), _F32),
        pltpu.VMEM((f, 1), _F32),
        pltpu.VMEM((f, n), _F32),
    ]

    body = functools.partial(_body, n, s, p['rs1_W'].shape[1], d_m)
    outs, am = pl.pallas_call(
        body,
        grid=(b, _C),
        in_specs=in_specs,
        out_specs=out_specs,
        out_shape=out_shape,
        scratch_shapes=scratch_shapes,
        compiler_params=pltpu.CompilerParams(
            dimension_semantics=("arbitrary", "arbitrary")),
    )(images, selfbbox, bbox_r, *weights)
    return outs, am.astype(bool)


# speedup vs baseline: 1.0396x; 1.0004x over previous
import functools

import jax
import jax.numpy as jnp
from jax import lax
from jax.experimental import pallas as pl
from jax.experimental.pallas import tpu as pltpu

_C = 4
_RELA_TH = 0.5

_F32 = jnp.float32


def _relu(x):
    return jnp.maximum(x, 0.0)


def _dot(a, b):
    return jnp.dot(a, b, preferred_element_type=_F32)


def _dot_t(a, b):
    return lax.dot_general(a, b, (((0,), (0,)), ((), ())),
                           preferred_element_type=_F32)


def _lnk(x, g, b, eps=1e-5):
    m = jnp.mean(x, axis=-1, keepdims=True)
    v = jnp.mean((x - m) ** 2, axis=-1, keepdims=True)
    return (x - m) / jnp.sqrt(v + eps) * g + b


_WNAMES = (
    'preW', 'preb', 'preg', 'prebe',
    'subjW', 'subjb', 'objW', 'objb',
    'ps1W', 'ps1b', 'psg', 'psbe', 'ps2W', 'ps2b',
    'pr1W', 'pr1b', 'prg', 'prbe', 'pr2W', 'pr2b',
    'rsfW', 'rsfb', 'rs2W', 'rs2b',
    'g1W', 'g1b', 'g2W', 'g2b', 'g3W', 'g3b',
)
_NW = len(_WNAMES)


def _body(n, s, d_rs, d_m, img_ref, sbb_ref, bbox_ref, *rest):
    f = s * n
    w = {name: rest[i] for i, name in enumerate(_WNAMES)}
    out_ref, am_ref = rest[_NW], rest[_NW + 1]
    (s_x, s_maskr, s_xsop, s_relas, s_A, s_mt, s_T) = rest[_NW + 2:]
    c = pl.program_id(1)

    @pl.when(c == 0)
    def _pernode():
        img = img_ref[0]
        rowsum = jnp.sum(img, axis=1, keepdims=True)
        mask = (rowsum != 0.0).astype(_F32)
        rr = lax.broadcasted_iota(jnp.int32, (n, n), 0)
        cc = lax.broadcasted_iota(jnp.int32, (n, n), 1)
        eye = (rr == cc).astype(_F32)
        s_maskr[...] = jnp.sum(eye * mask, axis=0, keepdims=True)
        am_ref[0, 0] = jnp.sum(eye * (1.0 - mask), axis=0, keepdims=True)

        x = _lnk(_relu(_dot(img, w['preW'][...]) + w['preb'][...]),
                 w['preg'][...], w['prebe'][...])
        s_x[...] = x
        xs = _relu(_dot(x, w['subjW'][...]) + w['subjb'][...]) * mask
        xo = _relu(_dot(x, w['objW'][...]) + w['objb'][...]) * mask
        sb = sbb_ref[0]
        t1 = _lnk(_relu(_dot(sb, w['ps1W'][...]) + w['ps1b'][...]),
                  w['psg'][...], w['psbe'][...])
        pself = _relu(_dot(t1, w['ps2W'][...]) + w['ps2b'][...]) * mask
        s_xsop[:n, :d_m] = xo
        s_xsop[:n, d_m:] = jnp.zeros_like(pself)
        s_xsop[n:, :d_m] = xs
        s_xsop[n:, d_m:] = pself
        s_relas[...] = jnp.zeros_like(s_relas)
        s_A[...] = jnp.zeros_like(s_A)

    s0 = c * s
    ii = lax.broadcasted_iota(jnp.int32, (f, 1), 0)
    tt = ii % n
    ss = ii // n + s0
    coln = lax.broadcasted_iota(jnp.int32, (f, n), 1)
    maskr = s_maskr[...]

    @pl.when(c == 0)
    def _t_once():
        T0 = (tt == coln).astype(_F32)
        s_T[...] = T0
        s_mt[...] = jnp.sum(T0 * maskr, axis=1, keepdims=True)

    T = s_T[...]
    Sx = (ss == coln).astype(_F32)
    dcol = (tt == ss).astype(_F32)
    mt = s_mt[...]
    ms = jnp.sum(Sx * maskr, axis=1, keepdims=True)

    bb = bbox_ref[0, 0]
    q = _lnk(_relu(_dot(bb, w['pr1W'][...]) + w['pr1b'][...]),
             w['prg'][...], w['prbe'][...])
    pp = _relu(_dot(q, w['pr2W'][...]) + w['pr2b'][...])
    sdidx = jnp.where(tt == ss, ss + n, tt)
    col2 = lax.broadcasted_iota(jnp.int32, (f, 2 * n), 1)
    SD = (sdidx == col2).astype(_F32)
    big = _dot(SD, s_xsop[...])
    posr = big[:, d_m:] + (1.0 - dcol) * pp
    feat = jnp.concatenate([big[:, :d_m], posr], axis=1)

    hp = _relu(_dot(feat, w['rsfW'][...]) + w['rsfb'][...])
    h = hp[:, :d_rs]
    proj = hp[:, d_rs:]
    logits = _dot(h, w['rs2W'][...]) + w['rs2b'][...]
    pfl = jax.nn.sigmoid(logits) * ms * mt
    afl = (pfl > _RELA_TH).astype(_F32)
    s_relas[...] += _dot_t(Sx, pfl * proj)
    s_A[...] += _dot_t(Sx, afl * T)

    @pl.when(c == _C - 1)
    def _gcn_out():
        x = s_x[...]
        relas = s_relas[...] * (1.0 / n)
        xx = jnp.concatenate([x, relas], axis=1)
        rr = lax.broadcasted_iota(jnp.int32, (n, n), 0)
        cc = lax.broadcasted_iota(jnp.int32, (n, n), 1)
        eye = (rr == cc).astype(_F32)
        A1 = jnp.maximum(s_A[...], eye)
        deg = jnp.sum(A1, axis=0, keepdims=True)
        dinv_r = jnp.where(deg > 0, 1.0 / jnp.sqrt(deg), 0.0)
        dinv_c = jnp.sum(eye * dinv_r, axis=1, keepdims=True)

        def gcn(y, W, b):
            yw = _dot(y, W)
            z = _dot_t(A1, dinv_c * yw)
            return dinv_c * z + b

        x1 = _relu(gcn(xx, w['g1W'][...], w['g1b'][...]))
        x2 = _relu(gcn(x1, w['g2W'][...], w['g2b'][...]))
        x3 = _relu(gcn(x2, w['g3W'][...], w['g3b'][...]))
        out_ref[0, 0] = x
        out_ref[0, 1] = x3


def kernel(images, selfbbox, bbox, params):
    b, n, d_in = images.shape
    s = n // _C
    f = s * n
    d_m = params['pre_W'].shape[1]
    p = params

    def row(v):
        return v.reshape(1, -1)

    weights = [
        p['pre_W'], row(p['pre_b']), row(p['pre_g']), row(p['pre_be']),
        p['subj_W'], row(p['subj_b']), p['obj_W'], row(p['obj_b']),
        p['ps1_W'], row(p['ps1_b']), row(p['ps_g']), row(p['ps_be']),
        p['ps2_W'], row(p['ps2_b']),
        p['pr1_W'], row(p['pr1_b']), row(p['pr_g']), row(p['pr_be']),
        p['pr2_W'], row(p['pr2_b']),
        jnp.concatenate([p['rs1_W'], p['rf_W']], axis=1),
        jnp.concatenate([row(p['rs1_b']), row(p['rf_b'])], axis=1),
        p['rs2_W'], p['rs2_b'].reshape(1, 1),
        p['g1_W'], row(p['g1_b']), p['g2_W'], row(p['g2_b']),
        p['g3_W'], row(p['g3_b']),
    ]
    bbox_r = bbox.reshape(b, _C, f, bbox.shape[-1])

    def const_spec(shape):
        nd = len(shape)
        return pl.BlockSpec(shape, lambda bi, ci, _nd=nd: (0,) * _nd)

    in_specs = [
        pl.BlockSpec((1, n, d_in), lambda bi, ci: (bi, 0, 0)),
        pl.BlockSpec((1, n, selfbbox.shape[-1]), lambda bi, ci: (bi, 0, 0)),
        pl.BlockSpec((1, 1, f, bbox.shape[-1]), lambda bi, ci: (bi, ci, 0, 0)),
    ] + [const_spec(wa.shape) for wa in weights]

    out_shape = [
        jax.ShapeDtypeStruct((b, 2, n, d_m), _F32),
        jax.ShapeDtypeStruct((b, 1, 1, n), _F32),
    ]
    out_specs = [
        pl.BlockSpec((1, 2, n, d_m), lambda bi, ci: (bi, 0, 0, 0)),
        pl.BlockSpec((1, 1, 1, n), lambda bi, ci: (bi, 0, 0, 0)),
    ]
    d_rf = p['rf_W'].shape[1]
    d_pair = p['rs1_W'].shape[0]
    scratch_shapes = [
        pltpu.VMEM((n, d_m), _F32),
        pltpu.VMEM((1, n), _F32),
        pltpu.VMEM((2 * n, d_pair), _F32),
        pltpu.VMEM((n, d_rf), _F32),
        pltpu.VMEM((n, n), _F32),
        pltpu.VMEM((f, 1), _F32),
        pltpu.VMEM((f, n), _F32),
    ]

    body = functools.partial(_body, n, s, p['rs1_W'].shape[1], d_m)
    outs, am = pl.pallas_call(
        body,
        grid=(b, _C),
        in_specs=in_specs,
        out_specs=out_specs,
        out_shape=out_shape,
        scratch_shapes=scratch_shapes,
        compiler_params=pltpu.CompilerParams(
            dimension_semantics=("arbitrary", "arbitrary")),
    )(images, selfbbox, bbox_r, *weights)
    return outs, am.astype(bool)
